# baseline (device time: 88519 ns/iter reference)
import jax
import jax.numpy as jnp
from jax import lax
from jax.experimental import pallas as pl
from jax.experimental.pallas import tpu as pltpu

N_DEV = 8
SQ = 1024
SKV_LOC = 1024
H = 8
DH = 128
D = H * DH
CH = SQ // N_DEV
BLK = 64
NB = SQ // BLK
SCALE = 0.08838834764831843
NEG = -1e9

CLASS_BLOCKS = [[0, 3, 6, 9, 12, 15], [1, 4, 7, 10, 13], [2, 5, 8, 11, 14]]
PACKED = [b for cls in CLASS_BLOCKS for b in cls]
INV = [PACKED.index(qb) for qb in range(NB)]
SEG1 = len(CLASS_BLOCKS[0]) * BLK
DIAG12 = CLASS_BLOCKS[1] + CLASS_BLOCKS[2]
ND12 = len(DIAG12)
R12 = ND12 * BLK
NV = 6 * BLK


def kernel(x, Wq, K_ext, V_ext, Wo):
    Qf = jnp.dot(x.reshape(SQ, D), Wq, preferred_element_type=jnp.float32)
    K2 = K_ext.reshape(SKV_LOC, D)
    V2 = V_ext.reshape(SKV_LOC, D)

    def body(q_ref, k_ref, v_ref, wo_ref, out_ref,
             o_loc, ml_loc, comm_o, comm_ml,
             send_o, send_ml, send_out, recv_o, recv_ml, recv_out, loc_sem):
        me = lax.axis_index("i")

        bsem = pltpu.get_barrier_semaphore()
        for j in range(N_DEV):
            @pl.when(j != me)
            def _(j=j):
                pl.semaphore_signal(bsem, inc=1, device_id=(j,),
                                    device_id_type=pl.DeviceIdType.MESH)
        pl.semaphore_wait(bsem, N_DEV - 1)

        def o_desc(peer, slot):
            return pltpu.make_async_remote_copy(
                src_ref=o_loc.at[pl.ds(peer * CH, CH), :],
                dst_ref=comm_o.at[slot],
                send_sem=send_o.at[peer],
                recv_sem=recv_o.at[slot],
                device_id=(peer,),
                device_id_type=pl.DeviceIdType.MESH)

        def ml_desc(peer, slot):
            return pltpu.make_async_remote_copy(
                src_ref=ml_loc.at[pl.ds(peer * CH, CH), :],
                dst_ref=comm_ml.at[slot],
                send_sem=send_ml.at[peer],
                recv_sem=recv_ml.at[slot],
                device_id=(peer,),
                device_id_type=pl.DeviceIdType.MESH)

        def out_desc(peer, row, sem):
            return pltpu.make_async_remote_copy(
                src_ref=out_ref.at[pl.ds(row * CH, CH), :],
                dst_ref=out_ref.at[pl.ds(row * CH, CH), :],
                send_sem=send_out.at[peer],
                recv_sem=recv_out.at[sem],
                device_id=(peer,),
                device_id_type=pl.DeviceIdType.MESH)

        def class_branch(cc):
            blocks = [b for b in range(NB) if b % 3 == cc]
            valid = len(blocks) * BLK

            def br(qr):
                kc = jnp.concatenate(
                    [k_ref[b * BLK:(b + 1) * BLK, :] for b in blocks], axis=0)
                vc = jnp.concatenate(
                    [v_ref[b * BLK:(b + 1) * BLK, :] for b in blocks], axis=0)
                if valid < NV:
                    pad = jnp.zeros((NV - valid, D), jnp.float32)
                    kc = jnp.concatenate([kc, pad], axis=0)
                    vc = jnp.concatenate([vc, pad], axis=0)
                cols = lax.broadcasted_iota(jnp.int32, (1, NV), 1)
                os_, ms_, ls_ = [], [], []
                for h in range(H):
                    s = lax.dot_general(
                        qr[:, h * DH:(h + 1) * DH], kc[:, h * DH:(h + 1) * DH],
                        (((1,), (1,)), ((), ())),
                        preferred_element_type=jnp.float32) * SCALE
                    if valid < NV:
                        s = jnp.where(cols < valid, s, NEG)
                    m = jnp.max(s, axis=1, keepdims=True)
                    w = jnp.exp(s - m)
                    lsum = jnp.sum(w, axis=1, keepdims=True)
                    o = lax.dot_general(
                        w, vc[:, h * DH:(h + 1) * DH],
                        (((1,), (0,)), ((), ())),
                        preferred_element_type=jnp.float32)
                    os_.append(o)
                    ms_.append(m)
                    ls_.append(lsum)
                return (jnp.concatenate(os_, axis=1),
                        jnp.concatenate(ms_, axis=1),
                        jnp.concatenate(ls_, axis=1))
            return br

        cls_res = []
        for r in range(3):
            qr = jnp.concatenate(
                [q_ref[b * BLK:(b + 1) * BLK, :] for b in CLASS_BLOCKS[r]],
                axis=0)
            c = (3 - (r + me) % 3) % 3
            cls_res.append(
                lax.switch(c, [class_branch(cc) for cc in range(3)], qr))

        o_loc[0:SEG1, :] = cls_res[0][0].astype(jnp.bfloat16)
        ml_loc[0:SEG1, 0:H] = cls_res[0][1]
        ml_loc[0:SEG1, H:2 * H] = cls_res[0][2]
        for j in range(SEG1 // CH):
            @pl.when(j != me)
            def _(j=j):
                o_desc(j, me).start()
                ml_desc(j, me).start()

        q12 = jnp.concatenate(
            [q_ref[b * BLK:(b + 1) * BLK, :] for b in DIAG12], axis=0)
        kd = jnp.concatenate(
            [k_ref[b * BLK:(b + 1) * BLK, :] for b in DIAG12],
            axis=0).reshape(ND12, BLK, D)
        vd = jnp.concatenate(
            [v_ref[b * BLK:(b + 1) * BLK, :] for b in DIAG12],
            axis=0).reshape(ND12, BLK, D)
        q12r = q12.reshape(ND12, BLK, D)
        on0 = me == 0
        for h in range(H):
            hc = slice(h * DH, (h + 1) * DH)
            sd = lax.dot_general(
                q12r[:, :, hc], kd[:, :, hc],
                (((2,), (2,)), ((0,), (0,))),
                preferred_element_type=jnp.float32) * SCALE
            s0 = lax.dot_general(
                q12[:, hc], k_ref[0:BLK, hc],
                (((1,), (1,)), ((), ())),
                preferred_element_type=jnp.float32) * SCALE
            sx = jnp.concatenate([sd.reshape(R12, BLK), s0], axis=1)
            sx = jnp.where(on0, sx, NEG)
            m_ex = jnp.max(sx, axis=1, keepdims=True)
            w_ex = jnp.exp(sx - m_ex)
            l_ex = jnp.sum(w_ex, axis=1, keepdims=True)
            o_ex = (lax.dot_general(
                        w_ex[:, 0:BLK].reshape(ND12, BLK, BLK), vd[:, :, hc],
                        (((2,), (1,)), ((0,), (0,))),
                        preferred_element_type=jnp.float32).reshape(R12, DH)
                    + lax.dot_general(
                        w_ex[:, BLK:2 * BLK], v_ref[0:BLK, hc],
                        (((1,), (0,)), ((), ())),
                        preferred_element_type=jnp.float32))
            m12 = jnp.concatenate(
                [cls_res[1][1][:, h:h + 1], cls_res[2][1][:, h:h + 1]], axis=0)
            l12 = jnp.concatenate(
                [cls_res[1][2][:, h:h + 1], cls_res[2][2][:, h:h + 1]], axis=0)
            o12 = jnp.concatenate(
                [cls_res[1][0][:, hc], cls_res[2][0][:, hc]], axis=0)
            mn = jnp.maximum(m12, m_ex)
            a = jnp.exp(m12 - mn)
            b = jnp.exp(m_ex - mn)
            o_loc[SEG1:SQ, hc] = (o12 * a + o_ex * b).astype(jnp.bfloat16)
            ml_loc[SEG1:SQ, h:h + 1] = mn
            ml_loc[SEG1:SQ, H + h:H + h + 1] = l12 * a + l_ex * b

        for j in range(SEG1 // CH, N_DEV):
            @pl.when(j != me)
            def _(j=j):
                o_desc(j, me).start()
                ml_desc(j, me).start()

        cp_o = pltpu.make_async_copy(
            o_loc.at[pl.ds(me * CH, CH), :], comm_o.at[me], loc_sem)
        cp_o.start()
        cp_o.wait()
        cp_ml = pltpu.make_async_copy(
            ml_loc.at[pl.ds(me * CH, CH), :], comm_ml.at[me], loc_sem)
        cp_ml.start()
        cp_ml.wait()

        for k in range(N_DEV):
            @pl.when(k != me)
            def _(k=k):
                o_desc(k, k).wait_recv()
                ml_desc(k, k).wait_recv()

        ctx_parts = []
        for h in range(H):
            m_acc = comm_ml[0, :, h:h + 1]
            l_acc = comm_ml[0, :, H + h:H + h + 1]
            o_acc = comm_o[0, :, h * DH:(h + 1) * DH]
            for k in range(1, N_DEV):
                mk = comm_ml[k, :, h:h + 1]
                lk = comm_ml[k, :, H + h:H + h + 1]
                ok = comm_o[k, :, h * DH:(h + 1) * DH]
                mn = jnp.maximum(m_acc, mk)
                a = jnp.exp(m_acc - mn)
                b = jnp.exp(mk - mn)
                o_acc = o_acc * a + ok * b
                l_acc = l_acc * a + lk * b
                m_acc = mn
            ctx_parts.append(o_acc / l_acc)
        ctx = jnp.concatenate(ctx_parts, axis=1)

        outc = jnp.dot(ctx, wo_ref[...], preferred_element_type=jnp.float32)
        out_ref[pl.ds(me * CH, CH), :] = outc.astype(jnp.bfloat16)

        for j in range(N_DEV):
            @pl.when(j != me)
            def _(j=j):
                out_desc(j, me, me).start()

        for k in range(N_DEV):
            @pl.when(k != me)
            def _(k=k):
                out_desc(k, k, k).wait_recv()

        for j in range(N_DEV):
            @pl.when(j != me)
            def _(j=j):
                o_desc(j, me).wait_send()
                ml_desc(j, me).wait_send()
                out_desc(j, me, me).wait_send()

    out2 = pl.pallas_call(
        body,
        out_shape=jax.ShapeDtypeStruct((SQ, D), jnp.bfloat16),
        in_specs=[pl.BlockSpec(memory_space=pltpu.VMEM)] * 4,
        out_specs=pl.BlockSpec(memory_space=pltpu.VMEM),
        scratch_shapes=[
            pltpu.VMEM((SQ, D), jnp.bfloat16),
            pltpu.VMEM((SQ, 2 * H), jnp.float32),
            pltpu.VMEM((N_DEV, CH, D), jnp.bfloat16),
            pltpu.VMEM((N_DEV, CH, 2 * H), jnp.float32),
            pltpu.SemaphoreType.DMA((N_DEV,)),
            pltpu.SemaphoreType.DMA((N_DEV,)),
            pltpu.SemaphoreType.DMA((N_DEV,)),
            pltpu.SemaphoreType.DMA((N_DEV,)),
            pltpu.SemaphoreType.DMA((N_DEV,)),
            pltpu.SemaphoreType.DMA((N_DEV,)),
            pltpu.SemaphoreType.DMA,
        ],
        compiler_params=pltpu.CompilerParams(
            collective_id=0, vmem_limit_bytes=60 * 1024 * 1024),
    )(Qf, K2, V2, Wo)
    out_nat = out2.reshape(NB, BLK, D)[jnp.array(INV)].reshape(SQ, D)
    return out_nat.astype(jnp.float32).reshape(1, SQ, D)


# device time: 86813 ns/iter; 1.0197x vs baseline; 1.0197x over previous
import jax
import jax.numpy as jnp
from jax import lax
from jax.experimental import pallas as pl
from jax.experimental.pallas import tpu as pltpu

N_DEV = 8
SQ = 1024
SKV_LOC = 1024
H = 8
DH = 128
D = H * DH
CH = SQ // N_DEV
BLK = 64
NB = SQ // BLK
SCALE = 0.08838834764831843
NEG = -1e9

CLASS_BLOCKS = [[0, 3, 6, 9, 12, 15], [1, 4, 7, 10, 13], [2, 5, 8, 11, 14]]
PACKED = [b for cls in CLASS_BLOCKS for b in cls]
INV = [PACKED.index(qb) for qb in range(NB)]
SEG1 = len(CLASS_BLOCKS[0]) * BLK
DIAG12 = CLASS_BLOCKS[1] + CLASS_BLOCKS[2]
ND12 = len(DIAG12)
R12 = ND12 * BLK
NV = 6 * BLK


def kernel(x, Wq, K_ext, V_ext, Wo):
    x2 = x.reshape(SQ, D)
    K2 = K_ext.reshape(SKV_LOC, D)
    V2 = V_ext.reshape(SKV_LOC, D)

    def body(x_ref, wq_ref, k_ref, v_ref, wo_ref, out_ref,
             o_loc, ml_loc, comm_o, comm_ml,
             send_o, send_ml, send_out, recv_o, recv_ml, recv_out, loc_sem):
        me = lax.axis_index("i")

        bsem = pltpu.get_barrier_semaphore()
        for j in range(N_DEV):
            @pl.when(j != me)
            def _(j=j):
                pl.semaphore_signal(bsem, inc=1, device_id=(j,),
                                    device_id_type=pl.DeviceIdType.MESH)
        pl.semaphore_wait(bsem, N_DEV - 1)

        def o_desc(peer, slot):
            return pltpu.make_async_remote_copy(
                src_ref=o_loc.at[pl.ds(peer * CH, CH), :],
                dst_ref=comm_o.at[slot],
                send_sem=send_o.at[peer],
                recv_sem=recv_o.at[slot],
                device_id=(peer,),
                device_id_type=pl.DeviceIdType.MESH)

        def ml_desc(peer, slot):
            return pltpu.make_async_remote_copy(
                src_ref=ml_loc.at[pl.ds(peer * CH, CH), :],
                dst_ref=comm_ml.at[slot],
                send_sem=send_ml.at[peer],
                recv_sem=recv_ml.at[slot],
                device_id=(peer,),
                device_id_type=pl.DeviceIdType.MESH)

        def out_desc(peer, row, sem):
            return pltpu.make_async_remote_copy(
                src_ref=out_ref.at[pl.ds(row * CH, CH), :],
                dst_ref=out_ref.at[pl.ds(row * CH, CH), :],
                send_sem=send_out.at[peer],
                recv_sem=recv_out.at[sem],
                device_id=(peer,),
                device_id_type=pl.DeviceIdType.MESH)

        def class_branch(cc):
            blocks = [b for b in range(NB) if b % 3 == cc]
            valid = len(blocks) * BLK

            def br(qr):
                kc = jnp.concatenate(
                    [k_ref[b * BLK:(b + 1) * BLK, :] for b in blocks], axis=0)
                vc = jnp.concatenate(
                    [v_ref[b * BLK:(b + 1) * BLK, :] for b in blocks], axis=0)
                if valid < NV:
                    pad = jnp.zeros((NV - valid, D), jnp.float32)
                    kc = jnp.concatenate([kc, pad], axis=0)
                    vc = jnp.concatenate([vc, pad], axis=0)
                cols = lax.broadcasted_iota(jnp.int32, (1, NV), 1)
                os_, ms_, ls_ = [], [], []
                for h in range(H):
                    s = lax.dot_general(
                        qr[:, h * DH:(h + 1) * DH], kc[:, h * DH:(h + 1) * DH],
                        (((1,), (1,)), ((), ())),
                        preferred_element_type=jnp.float32) * SCALE
                    if valid < NV:
                        s = jnp.where(cols < valid, s, NEG)
                    m = jnp.max(s, axis=1, keepdims=True)
                    w = jnp.exp(s - m)
                    lsum = jnp.sum(w, axis=1, keepdims=True)
                    o = lax.dot_general(
                        w, vc[:, h * DH:(h + 1) * DH],
                        (((1,), (0,)), ((), ())),
                        preferred_element_type=jnp.float32)
                    os_.append(o)
                    ms_.append(m)
                    ls_.append(lsum)
                return (jnp.concatenate(os_, axis=1),
                        jnp.concatenate(ms_, axis=1),
                        jnp.concatenate(ls_, axis=1))
            return br

        Qv = jnp.dot(x_ref[...], wq_ref[...],
                     preferred_element_type=jnp.float32)

        cls_res = []
        for r in range(3):
            qr = jnp.concatenate(
                [Qv[b * BLK:(b + 1) * BLK, :] for b in CLASS_BLOCKS[r]],
                axis=0)
            c = (3 - (r + me) % 3) % 3
            cls_res.append(
                lax.switch(c, [class_branch(cc) for cc in range(3)], qr))

        o_loc[0:SEG1, :] = cls_res[0][0].astype(jnp.bfloat16)
        ml_loc[0:SEG1, 0:H] = cls_res[0][1]
        ml_loc[0:SEG1, H:2 * H] = cls_res[0][2]
        for j in range(SEG1 // CH):
            @pl.when(j != me)
            def _(j=j):
                o_desc(j, me).start()
                ml_desc(j, me).start()

        q12 = jnp.concatenate(
            [Qv[b * BLK:(b + 1) * BLK, :] for b in DIAG12], axis=0)
        kd = jnp.concatenate(
            [k_ref[b * BLK:(b + 1) * BLK, :] for b in DIAG12],
            axis=0).reshape(ND12, BLK, D)
        vd = jnp.concatenate(
            [v_ref[b * BLK:(b + 1) * BLK, :] for b in DIAG12],
            axis=0).reshape(ND12, BLK, D)
        q12r = q12.reshape(ND12, BLK, D)
        on0 = me == 0
        for h in range(H):
            hc = slice(h * DH, (h + 1) * DH)
            sd = lax.dot_general(
                q12r[:, :, hc], kd[:, :, hc],
                (((2,), (2,)), ((0,), (0,))),
                preferred_element_type=jnp.float32) * SCALE
            s0 = lax.dot_general(
                q12[:, hc], k_ref[0:BLK, hc],
                (((1,), (1,)), ((), ())),
                preferred_element_type=jnp.float32) * SCALE
            sx = jnp.concatenate([sd.reshape(R12, BLK), s0], axis=1)
            sx = jnp.where(on0, sx, NEG)
            m_ex = jnp.max(sx, axis=1, keepdims=True)
            w_ex = jnp.exp(sx - m_ex)
            l_ex = jnp.sum(w_ex, axis=1, keepdims=True)
            o_ex = (lax.dot_general(
                        w_ex[:, 0:BLK].reshape(ND12, BLK, BLK), vd[:, :, hc],
                        (((2,), (1,)), ((0,), (0,))),
                        preferred_element_type=jnp.float32).reshape(R12, DH)
                    + lax.dot_general(
                        w_ex[:, BLK:2 * BLK], v_ref[0:BLK, hc],
                        (((1,), (0,)), ((), ())),
                        preferred_element_type=jnp.float32))
            m12 = jnp.concatenate(
                [cls_res[1][1][:, h:h + 1], cls_res[2][1][:, h:h + 1]], axis=0)
            l12 = jnp.concatenate(
                [cls_res[1][2][:, h:h + 1], cls_res[2][2][:, h:h + 1]], axis=0)
            o12 = jnp.concatenate(
                [cls_res[1][0][:, hc], cls_res[2][0][:, hc]], axis=0)
            mn = jnp.maximum(m12, m_ex)
            a = jnp.exp(m12 - mn)
            b = jnp.exp(m_ex - mn)
            o_loc[SEG1:SQ, hc] = (o12 * a + o_ex * b).astype(jnp.bfloat16)
            ml_loc[SEG1:SQ, h:h + 1] = mn
            ml_loc[SEG1:SQ, H + h:H + h + 1] = l12 * a + l_ex * b

        for j in range(SEG1 // CH, N_DEV):
            @pl.when(j != me)
            def _(j=j):
                o_desc(j, me).start()
                ml_desc(j, me).start()

        cp_o = pltpu.make_async_copy(
            o_loc.at[pl.ds(me * CH, CH), :], comm_o.at[me], loc_sem)
        cp_o.start()
        cp_o.wait()
        cp_ml = pltpu.make_async_copy(
            ml_loc.at[pl.ds(me * CH, CH), :], comm_ml.at[me], loc_sem)
        cp_ml.start()
        cp_ml.wait()

        for k in range(N_DEV):
            @pl.when(k != me)
            def _(k=k):
                o_desc(k, k).wait_recv()
                ml_desc(k, k).wait_recv()

        ctx_parts = []
        for h in range(H):
            m_acc = comm_ml[0, :, h:h + 1]
            l_acc = comm_ml[0, :, H + h:H + h + 1]
            o_acc = comm_o[0, :, h * DH:(h + 1) * DH]
            for k in range(1, N_DEV):
                mk = comm_ml[k, :, h:h + 1]
                lk = comm_ml[k, :, H + h:H + h + 1]
                ok = comm_o[k, :, h * DH:(h + 1) * DH]
                mn = jnp.maximum(m_acc, mk)
                a = jnp.exp(m_acc - mn)
                b = jnp.exp(mk - mn)
                o_acc = o_acc * a + ok * b
                l_acc = l_acc * a + lk * b
                m_acc = mn
            ctx_parts.append(o_acc / l_acc)
        ctx = jnp.concatenate(ctx_parts, axis=1)

        outc = jnp.dot(ctx, wo_ref[...], preferred_element_type=jnp.float32)
        out_ref[pl.ds(me * CH, CH), :] = outc.astype(jnp.bfloat16)

        for j in range(N_DEV):
            @pl.when(j != me)
            def _(j=j):
                out_desc(j, me, me).start()

        for k in range(N_DEV):
            @pl.when(k != me)
            def _(k=k):
                out_desc(k, k, k).wait_recv()

        for j in range(N_DEV):
            @pl.when(j != me)
            def _(j=j):
                o_desc(j, me).wait_send()
                ml_desc(j, me).wait_send()
                out_desc(j, me, me).wait_send()

    out2 = pl.pallas_call(
        body,
        out_shape=jax.ShapeDtypeStruct((SQ, D), jnp.bfloat16),
        in_specs=[pl.BlockSpec(memory_space=pltpu.VMEM)] * 5,
        out_specs=pl.BlockSpec(memory_space=pltpu.VMEM),
        scratch_shapes=[
            pltpu.VMEM((SQ, D), jnp.bfloat16),
            pltpu.VMEM((SQ, 2 * H), jnp.float32),
            pltpu.VMEM((N_DEV, CH, D), jnp.bfloat16),
            pltpu.VMEM((N_DEV, CH, 2 * H), jnp.float32),
            pltpu.SemaphoreType.DMA((N_DEV,)),
            pltpu.SemaphoreType.DMA((N_DEV,)),
            pltpu.SemaphoreType.DMA((N_DEV,)),
            pltpu.SemaphoreType.DMA((N_DEV,)),
            pltpu.SemaphoreType.DMA((N_DEV,)),
            pltpu.SemaphoreType.DMA((N_DEV,)),
            pltpu.SemaphoreType.DMA,
        ],
        compiler_params=pltpu.CompilerParams(
            collective_id=0, vmem_limit_bytes=60 * 1024 * 1024),
    )(x2, Wq, K2, V2, Wo)
    out_nat = out2.reshape(NB, BLK, D)[jnp.array(INV)].reshape(SQ, D)
    return out_nat.astype(jnp.float32).reshape(1, SQ, D)
